# baseline (speedup 1.0000x reference)
import jax
import jax.numpy as jnp
from jax import lax
from jax.experimental import pallas as pl
from jax.experimental.pallas import tpu as pltpu
from jax.experimental.pallas import tpu_sc as plsc

_B, _N, _K = 64, 32768, 2048
_NC, _NS = 2, 16
_NW = _NC * _NS
_RW = _B // _NW
_L = 16
_STEPS = _RW * _K // _L
_RSHIFT = (_K // _L).bit_length() - 1
_NSHIFT = _N.bit_length() - 1


def _gather_body(x_hbm, idx_hbm, out_hbm, x_v, i_v, o_v, sem_x, sem_i):
    wid = lax.axis_index("s") * _NC + lax.axis_index("c")
    dx = pltpu.async_copy(x_hbm.at[wid], x_v, sem_x)
    di = pltpu.async_copy(idx_hbm.at[wid], i_v, sem_i)
    di.wait()
    dx.wait()

    def step(i, carry):
        off = lax.shift_left(lax.shift_right_logical(i, _RSHIFT), _NSHIFT)
        iv = i_v[pl.ds(i * _L, _L)] + off
        o_v[pl.ds(i * _L, _L)] = plsc.load_gather(x_v, [iv])
        return carry

    lax.fori_loop(0, _STEPS, step, 0, unroll=4)
    pltpu.sync_copy(o_v, out_hbm.at[wid])


@jax.jit
def _run(x, idx):
    mesh = plsc.VectorSubcoreMesh(core_axis_name="c", subcore_axis_name="s")
    f = pl.kernel(
        _gather_body,
        out_type=jax.ShapeDtypeStruct((_NW, _RW * _K), jnp.float32),
        mesh=mesh,
        scratch_types=[
            pltpu.VMEM((_RW * _N,), jnp.float32),
            pltpu.VMEM((_RW * _K,), jnp.int32),
            pltpu.VMEM((_RW * _K,), jnp.float32),
            pltpu.SemaphoreType.DMA,
            pltpu.SemaphoreType.DMA,
        ],
        compiler_params=pltpu.CompilerParams(needs_layout_passes=False),
    )
    return f(x.reshape(_NW, _RW * _N), idx.reshape(_NW, _RW * _K))


def kernel(x, assessment, index):
    del assessment
    return _run(x, index.astype(jnp.int32)).reshape(_B, _K)


# speedup vs baseline: 1.3691x; 1.3691x over previous
import jax
import jax.numpy as jnp
from jax import lax
from jax.experimental import pallas as pl
from jax.experimental.pallas import tpu as pltpu
from jax.experimental.pallas import tpu_sc as plsc

_B, _N, _K = 64, 32768, 2048
_NC, _NS = 2, 16
_NW = _NC * _NS
_RW = _B // _NW
_L = 16
_STEPS = _K // _L


def _gather_body(x_hbm, idx_hbm, out_hbm,
                 x0_v, x1_v, i0_v, i1_v, o0_v, o1_v,
                 sem_a, sem_b, sem_o):
    wid = lax.axis_index("s") * _NC + lax.axis_index("c")
    row0 = wid * _RW
    row1 = row0 + 1

    di0 = pltpu.async_copy(idx_hbm.at[row0], i0_v, sem_a)
    di1 = pltpu.async_copy(idx_hbm.at[row1], i1_v, sem_b)
    dx0 = pltpu.async_copy(x_hbm.at[row0], x0_v, sem_a)
    dx1 = pltpu.async_copy(x_hbm.at[row1], x1_v, sem_b)

    def gather_row(x_v, idx_v, out_v):
        def step(i, carry):
            iv = idx_v[pl.ds(i * _L, _L)]
            out_v[pl.ds(i * _L, _L)] = plsc.load_gather(x_v, [iv])
            return carry

        lax.fori_loop(0, _STEPS, step, 0, unroll=8)

    di0.wait()
    dx0.wait()
    gather_row(x0_v, i0_v, o0_v)
    do0 = pltpu.async_copy(o0_v, out_hbm.at[row0], sem_o)
    di1.wait()
    dx1.wait()
    gather_row(x1_v, i1_v, o1_v)
    do1 = pltpu.async_copy(o1_v, out_hbm.at[row1], sem_o)
    do0.wait()
    do1.wait()


@jax.jit
def _run(x, idx):
    mesh = plsc.VectorSubcoreMesh(core_axis_name="c", subcore_axis_name="s")
    f = pl.kernel(
        _gather_body,
        out_type=jax.ShapeDtypeStruct((_B, _K), jnp.float32),
        mesh=mesh,
        scratch_types=[
            pltpu.VMEM((_N,), jnp.float32),
            pltpu.VMEM((_N,), jnp.float32),
            pltpu.VMEM((_K,), jnp.int32),
            pltpu.VMEM((_K,), jnp.int32),
            pltpu.VMEM((_K,), jnp.float32),
            pltpu.VMEM((_K,), jnp.float32),
            pltpu.SemaphoreType.DMA,
            pltpu.SemaphoreType.DMA,
            pltpu.SemaphoreType.DMA,
        ],
        compiler_params=pltpu.CompilerParams(needs_layout_passes=False),
    )
    return f(x, idx)


def kernel(x, assessment, index):
    del assessment
    return _run(x, index.astype(jnp.int32))


# speedup vs baseline: 1.3731x; 1.0030x over previous
import jax
import jax.numpy as jnp
from jax import lax
from jax.experimental import pallas as pl
from jax.experimental.pallas import tpu as pltpu
from jax.experimental.pallas import tpu_sc as plsc

_B, _N, _K = 64, 32768, 2048
_NC, _NS = 2, 16
_NW = _NC * _NS
_RW = _B // _NW
_L = 16
_STEPS = _K // _L


def _gather_body(x_hbm, idx_hbm, out_hbm,
                 x0_v, x1_v, i0_v, i1_v, o0_v, o1_v,
                 sem_a, sem_b, sem_o):
    wid = lax.axis_index("s") * _NC + lax.axis_index("c")
    row0 = wid * _RW
    row1 = row0 + 1

    di0 = pltpu.async_copy(idx_hbm.at[row0], i0_v, sem_a)
    di1 = pltpu.async_copy(idx_hbm.at[row1], i1_v, sem_b)
    dx0 = pltpu.async_copy(x_hbm.at[row0], x0_v, sem_a)
    dx1 = pltpu.async_copy(x_hbm.at[row1], x1_v, sem_b)

    def gather_row(x_v, idx_v, out_v):
        def step(i, carry):
            iv = idx_v[pl.ds(i * _L, _L)]
            out_v[pl.ds(i * _L, _L)] = plsc.load_gather(x_v, [iv])
            return carry

        lax.fori_loop(0, _STEPS, step, 0, unroll=16)

    di0.wait()
    dx0.wait()
    gather_row(x0_v, i0_v, o0_v)
    do0 = pltpu.async_copy(o0_v, out_hbm.at[row0], sem_o)
    di1.wait()
    dx1.wait()
    gather_row(x1_v, i1_v, o1_v)
    do1 = pltpu.async_copy(o1_v, out_hbm.at[row1], sem_o)
    do0.wait()
    do1.wait()


@jax.jit
def _run(x, idx):
    mesh = plsc.VectorSubcoreMesh(core_axis_name="c", subcore_axis_name="s")
    f = pl.kernel(
        _gather_body,
        out_type=jax.ShapeDtypeStruct((_B, _K), jnp.float32),
        mesh=mesh,
        scratch_types=[
            pltpu.VMEM((_N,), jnp.float32),
            pltpu.VMEM((_N,), jnp.float32),
            pltpu.VMEM((_K,), jnp.int32),
            pltpu.VMEM((_K,), jnp.int32),
            pltpu.VMEM((_K,), jnp.float32),
            pltpu.VMEM((_K,), jnp.float32),
            pltpu.SemaphoreType.DMA,
            pltpu.SemaphoreType.DMA,
            pltpu.SemaphoreType.DMA,
        ],
        compiler_params=pltpu.CompilerParams(needs_layout_passes=False),
    )
    return f(x, idx)


def kernel(x, assessment, index):
    del assessment
    return _run(x, index.astype(jnp.int32))
